# Initial kernel scaffold; baseline (speedup 1.0000x reference)
import functools

import jax
import jax.numpy as jnp
from jax import lax
from jax.experimental import pallas as pl
from jax.experimental.pallas import tpu as pltpu
from jax.experimental.pallas import tpu_sc as plsc

B, C, H, W = 8, 96, 112, 112
K = 2
NH, NW_ = H * K, W * K
PLANES = B * C
IN_PLANE = H * W
OUT_PLANE = NH * NW_
N_WORKERS = 32
PER_WORKER = PLANES // N_WORKERS
LANES = 16
CHUNKS_PER_ROW = W // LANES

_mesh = plsc.VectorSubcoreMesh(core_axis_name="c", subcore_axis_name="s")


@functools.partial(
    pl.kernel,
    out_type=jax.ShapeDtypeStruct((PLANES, OUT_PLANE), jnp.float32),
    mesh=_mesh,
    scratch_types=[
        pltpu.VMEM((IN_PLANE,), jnp.float32),
        pltpu.VMEM((OUT_PLANE,), jnp.float32),
    ],
)
def _unpool_sc(in_hbm, out_hbm, in_v, out_v):
    wid = lax.axis_index("s") * 2 + lax.axis_index("c")

    zero = jnp.zeros((LANES,), jnp.float32)

    def zbody(t, carry):
        out_v[pl.ds(t * LANES, LANES)] = zero
        return carry

    lax.fori_loop(0, OUT_PLANE // LANES, zbody, 0)

    two_iota = lax.iota(jnp.int32, LANES) * 2

    def plane_body(p, carry):
        plane = wid * PER_WORKER + p
        pltpu.sync_copy(in_hbm.at[plane], in_v)

        def row_body(i, c2):
            base = (2 * i + 1) * NW_ + 1
            idx0 = two_iota + base
            for q in range(CHUNKS_PER_ROW):
                x = in_v[pl.ds(i * W + q * LANES, LANES)]
                plsc.store_scatter(out_v, [idx0 + q * 2 * LANES], x)
            return c2

        lax.fori_loop(0, H, row_body, 0)
        pltpu.sync_copy(out_v, out_hbm.at[plane])
        return carry

    lax.fori_loop(0, PER_WORKER, plane_body, 0)


def kernel(input):
    flat = input.reshape(PLANES, IN_PLANE)
    out = _unpool_sc(flat)
    return out.reshape(B, C, NH, NW_)


# speedup vs baseline: 4.2994x; 4.2994x over previous
import functools

import jax
import jax.numpy as jnp
from jax import lax
from jax.experimental import pallas as pl
from jax.experimental.pallas import tpu as pltpu
from jax.experimental.pallas import tpu_sc as plsc

B, C, H, W = 8, 96, 112, 112
K = 2
NH, NW_ = H * K, W * K
PLANES = B * C
IN_PLANE = H * W
OUT_PLANE = NH * NW_
N_WORKERS = 32
PER_WORKER = PLANES // N_WORKERS
LANES = 16
CHUNKS_PER_ROW = W // LANES

_mesh = plsc.VectorSubcoreMesh(core_axis_name="c", subcore_axis_name="s")


@functools.partial(
    pl.kernel,
    out_type=jax.ShapeDtypeStruct((PLANES, OUT_PLANE), jnp.float32),
    mesh=_mesh,
    scratch_types=[
        pltpu.VMEM((IN_PLANE,), jnp.float32),
        pltpu.VMEM((OUT_PLANE,), jnp.float32),
    ],
    compiler_params=pltpu.CompilerParams(needs_layout_passes=False),
)
def _unpool_sc(in_hbm, out_hbm, in_v, out_v):
    wid = lax.axis_index("s") * 2 + lax.axis_index("c")

    zero = jnp.zeros((LANES,), jnp.float32)

    def zbody(t, carry):
        out_v[pl.ds(t * LANES, LANES)] = zero
        return carry

    lax.fori_loop(0, OUT_PLANE // LANES, zbody, 0)

    two_iota = lax.iota(jnp.int32, LANES) * 2

    def plane_body(p, carry):
        plane = wid * PER_WORKER + p
        pltpu.sync_copy(in_hbm.at[plane], in_v)

        def row_body(i, c2):
            base = (2 * i + 1) * NW_ + 1
            idx0 = two_iota + base
            for q in range(CHUNKS_PER_ROW):
                x = in_v[pl.ds(i * W + q * LANES, LANES)]
                plsc.store_scatter(out_v, [idx0 + q * 2 * LANES], x)
            return c2

        lax.fori_loop(0, H, row_body, 0)
        pltpu.sync_copy(out_v, out_hbm.at[plane])
        return carry

    lax.fori_loop(0, PER_WORKER, plane_body, 0)


def kernel(input):
    flat = input.reshape(PLANES, IN_PLANE)
    out = _unpool_sc(flat)
    return out.reshape(B, C, NH, NW_)


# speedup vs baseline: 5.1522x; 1.1983x over previous
import functools

import jax
import jax.numpy as jnp
from jax import lax
from jax.experimental import pallas as pl
from jax.experimental.pallas import tpu as pltpu
from jax.experimental.pallas import tpu_sc as plsc

B, C, H, W = 8, 96, 112, 112
K = 2
NH, NW_ = H * K, W * K
PLANES = B * C
IN_PLANE = H * W
OUT_PLANE = NH * NW_
N_WORKERS = 32
PER_WORKER = PLANES // N_WORKERS
LANES = 16
CHUNKS_PER_ROW = W // LANES

_mesh = plsc.VectorSubcoreMesh(core_axis_name="c", subcore_axis_name="s")


def _zero_fill(out_v):
    zero = jnp.zeros((LANES,), jnp.float32)

    def zbody(t, carry):
        out_v[pl.ds(t * LANES, LANES)] = zero
        return carry

    lax.fori_loop(0, OUT_PLANE // LANES, zbody, 0)


def _scatter_plane(in_v, out_v):
    two_iota = lax.iota(jnp.int32, LANES) * 2

    def row_body(i, carry):
        base = (2 * i + 1) * NW_ + 1
        idx0 = two_iota + base
        for q in range(CHUNKS_PER_ROW):
            x = in_v[pl.ds(i * W + q * LANES, LANES)]
            plsc.store_scatter(out_v, [idx0 + q * 2 * LANES], x)
        return carry

    lax.fori_loop(0, H, row_body, 0)


@functools.partial(
    pl.kernel,
    out_type=jax.ShapeDtypeStruct((PLANES, OUT_PLANE), jnp.float32),
    mesh=_mesh,
    scratch_types=[
        pltpu.VMEM((IN_PLANE,), jnp.float32),
        pltpu.VMEM((IN_PLANE,), jnp.float32),
        pltpu.VMEM((OUT_PLANE,), jnp.float32),
        pltpu.VMEM((OUT_PLANE,), jnp.float32),
        pltpu.SemaphoreType.DMA,
        pltpu.SemaphoreType.DMA,
        pltpu.SemaphoreType.DMA,
        pltpu.SemaphoreType.DMA,
    ],
    compiler_params=pltpu.CompilerParams(needs_layout_passes=False),
)
def _unpool_sc(in_hbm, out_hbm, in_v0, in_v1, out_v0, out_v1,
               sem_i0, sem_i1, sem_o0, sem_o1):
    wid = lax.axis_index("s") * 2 + lax.axis_index("c")
    base_plane = wid * PER_WORKER

    in_v = [in_v0, in_v1]
    out_v = [out_v0, out_v1]
    sem_i = [sem_i0, sem_i1]
    sem_o = [sem_o0, sem_o1]

    _zero_fill(out_v0)
    _zero_fill(out_v1)

    in_descs = [None, None]
    out_descs = [None, None]
    in_descs[0] = pltpu.async_copy(in_hbm.at[base_plane], in_v[0], sem_i[0])
    for p in range(PER_WORKER):
        b = p % 2
        if p + 1 < PER_WORKER:
            nb = (p + 1) % 2
            in_descs[nb] = pltpu.async_copy(
                in_hbm.at[base_plane + p + 1], in_v[nb], sem_i[nb])
        in_descs[b].wait()
        if p >= 2:
            out_descs[b].wait()
        _scatter_plane(in_v[b], out_v[b])
        out_descs[b] = pltpu.async_copy(
            out_v[b], out_hbm.at[base_plane + p], sem_o[b])
    out_descs[(PER_WORKER - 2) % 2].wait()
    out_descs[(PER_WORKER - 1) % 2].wait()


def kernel(input):
    flat = input.reshape(PLANES, IN_PLANE)
    out = _unpool_sc(flat)
    return out.reshape(B, C, NH, NW_)


# speedup vs baseline: 9.4518x; 1.8345x over previous
import functools

import jax
import jax.numpy as jnp
from jax import lax
from jax.experimental import pallas as pl
from jax.experimental.pallas import tpu as pltpu
from jax.experimental.pallas import tpu_sc as plsc

B, C, H, W = 8, 96, 112, 112
K = 2
NH, NW_ = H * K, W * K
PLANES = B * C
N_WORKERS = 32
PER_WORKER = PLANES // N_WORKERS
LANES = 16

_mesh = plsc.VectorSubcoreMesh(core_axis_name="c", subcore_axis_name="s")


@functools.partial(
    pl.kernel,
    out_type=jax.ShapeDtypeStruct((B, C, NH, NW_), jnp.float32),
    mesh=_mesh,
    scratch_types=[
        pltpu.VMEM((H, W), jnp.float32),
        pltpu.VMEM((NH, NW_), jnp.float32),
    ],
    compiler_params=pltpu.CompilerParams(
        needs_layout_passes=False, use_tc_tiling_on_sc=True),
)
def _unpool_sc(in_hbm, out_hbm, in_v, out_v):
    wid = lax.axis_index("s") * 2 + lax.axis_index("c")
    base_plane = wid * PER_WORKER

    zero = jnp.zeros((LANES,), jnp.float32)

    def zrow(r, carry):
        def zcol(t, c2):
            out_v[r, pl.ds(t * LANES, LANES)] = zero
            return c2
        lax.fori_loop(0, NW_ // LANES, zcol, carry)
        return carry

    lax.fori_loop(0, NH, zrow, 0)

    two_iota = lax.iota(jnp.int32, LANES) * 2

    def plane_body(p, carry):
        plane = base_plane + p
        b = plane // C
        c = plane % C
        pltpu.sync_copy(in_hbm.at[b, c], in_v)

        def row_body(i, c2):
            row_idx = jnp.full((LANES,), 2 * i + 1, jnp.int32)
            for q in range(W // LANES):
                x = in_v[i, pl.ds(q * LANES, LANES)]
                col_idx = two_iota + (2 * q * LANES + 1)
                plsc.store_scatter(out_v, [row_idx, col_idx], x)
            return c2

        lax.fori_loop(0, H, row_body, 0)
        pltpu.sync_copy(out_v, out_hbm.at[b, c])
        return carry

    lax.fori_loop(0, PER_WORKER, plane_body, 0)


def kernel(input):
    return _unpool_sc(input)


# speedup vs baseline: 14.7081x; 1.5561x over previous
import functools

import jax
import jax.numpy as jnp
from jax import lax
from jax.experimental import pallas as pl
from jax.experimental.pallas import tpu as pltpu
from jax.experimental.pallas import tpu_sc as plsc

B, C, H, W = 8, 96, 112, 112
K = 2
NH, NW_ = H * K, W * K
PLANES = B * C
LANES = 16

RIN = 56
ROUT = RIN * K
SPLITS = H // RIN
UNITS = PLANES * SPLITS
N_WORKERS = 32
PER_WORKER = UNITS // N_WORKERS

_mesh = plsc.VectorSubcoreMesh(core_axis_name="c", subcore_axis_name="s")


def _zero_fill(out_v):
    zero = jnp.zeros((LANES,), jnp.float32)

    def zrow(r, carry):
        def zcol(t, c2):
            out_v[r, pl.ds(t * LANES, LANES)] = zero
            return c2
        return lax.fori_loop(0, NW_ // LANES, zcol, carry)

    lax.fori_loop(0, ROUT, zrow, 0)


def _scatter_block(in_v, out_v):
    two_iota = lax.iota(jnp.int32, LANES) * 2

    def row_body(i, carry):
        row_idx = jnp.full((LANES,), 2 * i + 1, jnp.int32)
        for q in range(W // LANES):
            x = in_v[i, pl.ds(q * LANES, LANES)]
            col_idx = two_iota + (2 * q * LANES + 1)
            plsc.store_scatter(out_v, [row_idx, col_idx], x)
        return carry

    lax.fori_loop(0, RIN, row_body, 0)


@functools.partial(
    pl.kernel,
    out_type=jax.ShapeDtypeStruct((B, C, NH, NW_), jnp.float32),
    mesh=_mesh,
    scratch_types=[
        pltpu.VMEM((RIN, W), jnp.float32),
        pltpu.VMEM((RIN, W), jnp.float32),
        pltpu.VMEM((ROUT, NW_), jnp.float32),
        pltpu.VMEM((ROUT, NW_), jnp.float32),
        pltpu.SemaphoreType.DMA,
        pltpu.SemaphoreType.DMA,
        pltpu.SemaphoreType.DMA,
        pltpu.SemaphoreType.DMA,
    ],
    compiler_params=pltpu.CompilerParams(
        needs_layout_passes=False, use_tc_tiling_on_sc=True),
)
def _unpool_sc(in_hbm, out_hbm, in_v0, in_v1, out_v0, out_v1,
               sem_i0, sem_i1, sem_o0, sem_o1):
    wid = lax.axis_index("s") * 2 + lax.axis_index("c")
    base_unit = wid * PER_WORKER

    in_v = [in_v0, in_v1]
    out_v = [out_v0, out_v1]
    sem_i = [sem_i0, sem_i1]
    sem_o = [sem_o0, sem_o1]

    _zero_fill(out_v0)
    _zero_fill(out_v1)

    def in_slice(u):
        unit = base_unit + u
        plane = unit // SPLITS
        half = unit % SPLITS
        return in_hbm.at[plane // C, plane % C, pl.ds(half * RIN, RIN)]

    def out_slice(u):
        unit = base_unit + u
        plane = unit // SPLITS
        half = unit % SPLITS
        return out_hbm.at[plane // C, plane % C, pl.ds(half * ROUT, ROUT)]

    in_descs = [None, None]
    out_descs = [None, None]
    in_descs[0] = pltpu.async_copy(in_slice(0), in_v[0], sem_i[0])
    for p in range(PER_WORKER):
        u = p % 2
        if p + 1 < PER_WORKER:
            nu = (p + 1) % 2
            in_descs[nu] = pltpu.async_copy(in_slice(p + 1), in_v[nu], sem_i[nu])
        in_descs[u].wait()
        if p >= 2:
            out_descs[u].wait()
        _scatter_block(in_v[u], out_v[u])
        out_descs[u] = pltpu.async_copy(out_v[u], out_slice(p), sem_o[u])
    out_descs[(PER_WORKER - 2) % 2].wait()
    out_descs[(PER_WORKER - 1) % 2].wait()


def kernel(input):
    return _unpool_sc(input)
